# baseline (speedup 1.0000x reference)
import functools
import jax
import jax.numpy as jnp
from jax import lax
from jax.experimental import pallas as pl
from jax.experimental.pallas import tpu as pltpu
from jax.experimental.pallas import tpu_sc as plsc

SINK = 256
RECENT = 4096
TOP_C = 2048
TCAP = 32768
OLD_END = TCAP - RECENT
KEEP = SINK + TOP_C + RECENT
D = 2048
S_CHUNK = 32
S_NCHUNK = 8
ZROWS = 48
TAIL = TCAP - KEEP
TAIL_PER_TILE = TAIL // 32

C0_LO = 1280
C0_HI_BASE = 3200
C0_HI_END = 5248


def _sc_compact(mem_k2, mem_v2, new_k2, new_v2, sel_hbm, jl_hbm, pl_hbm,
                cnt_hbm, zero_hbm):
    mesh = plsc.VectorSubcoreMesh(core_axis_name="c", subcore_axis_name="s",
                                  num_cores=2, num_subcores=16)

    @functools.partial(
        pl.kernel,
        out_type=(
            jax.ShapeDtypeStruct((TCAP, D), jnp.float32),
            jax.ShapeDtypeStruct((TCAP, D), jnp.float32),
        ),
        mesh=mesh,
        scratch_types=[
            pltpu.VMEM((ZROWS, D), jnp.float32),
            pltpu.VMEM((24,), jnp.int32),
            pltpu.VMEM((24,), jnp.int32),
            pltpu.VMEM((16,), jnp.int32),
            pltpu.VMEM((S_CHUNK,), jnp.int32),
            pltpu.VMEM((S_CHUNK,), jnp.int32),
            pltpu.VMEM((16,), jnp.int32),
            pltpu.SemaphoreType.DMA,
            pltpu.SemaphoreType.DMA,
            pltpu.SemaphoreType.DMA,
            pltpu.SemaphoreType.DMA,
        ],
    )
    def k(mem_k, mem_v, new_k, new_v, sel, jl, plst, cnts, zsrc,
          out_k, out_v,
          buf, ia, ib, ic, jv, pv, cntv, sgk, sgv, ssk, ssv):
        core = lax.axis_index("c")
        sub = lax.axis_index("s")
        wid = core * 16 + sub

        selbase = jnp.where(core == 0, 0, 1024) + sub * 64
        outbase = jnp.where(core == 0, SINK, C0_LO) + sub * 64
        chunks = [(0, 24, ia), (24, 24, ib), (48, 16, ic)]
        prev = None
        for off, sz, iref in chunks:
            pltpu.sync_copy(sel.at[pl.ds(selbase + off, sz)], iref)
            if prev is not None:
                prev[0].wait()
                prev[1].wait()
            gk = pltpu.async_copy(mem_k.at[iref], buf.at[pl.ds(0, sz)], sgk)
            gv = pltpu.async_copy(mem_v.at[iref], buf.at[pl.ds(24, sz)], sgv)
            gk.wait()
            gv.wait()
            sk = pltpu.async_copy(buf.at[pl.ds(0, sz)],
                                  out_k.at[pl.ds(outbase + off, sz)], ssk)
            sv = pltpu.async_copy(buf.at[pl.ds(24, sz)],
                                  out_v.at[pl.ds(outbase + off, sz)], ssv)
            prev = (sk, sv)
        prev[0].wait()
        prev[1].wait()

        @pl.when(core == 0)
        def _():
            sb = sub * 16
            pltpu.sync_copy(mem_k.at[pl.ds(sb, 16)], out_k.at[pl.ds(sb, 16)])
            pltpu.sync_copy(mem_v.at[pl.ds(sb, 16)], out_v.at[pl.ds(sb, 16)])
            ra = sub * 128
            pltpu.sync_copy(mem_k.at[pl.ds(29568 + ra, 128)],
                            out_k.at[pl.ds(C0_HI_BASE + ra, 128)])
            pltpu.sync_copy(mem_v.at[pl.ds(29568 + ra, 128)],
                            out_v.at[pl.ds(C0_HI_BASE + ra, 128)])

        @pl.when(core == 1)
        def _():
            rb = sub * 56
            pltpu.sync_copy(mem_k.at[pl.ds(OLD_END + rb, 56)],
                            out_k.at[pl.ds(SINK + TOP_C + rb, 56)])
            pltpu.sync_copy(mem_v.at[pl.ds(OLD_END + rb, 56)],
                            out_v.at[pl.ds(SINK + TOP_C + rb, 56)])
            rc = sub * 72
            pltpu.sync_copy(mem_k.at[pl.ds(31616 + rc, 72)],
                            out_k.at[pl.ds(C0_HI_END + rc, 72)])
            pltpu.sync_copy(mem_v.at[pl.ds(31616 + rc, 72)],
                            out_v.at[pl.ds(C0_HI_END + rc, 72)])

        plsc.subcore_barrier()

        pltpu.sync_copy(cnts.at[pl.ds(wid * 16, 16)], cntv)
        ncv = cntv[...][0]

        def sbody(c, carry):
            soff = (wid * S_NCHUNK + c) * S_CHUNK
            pltpu.sync_copy(jl.at[pl.ds(soff, S_CHUNK)], jv)
            pltpu.sync_copy(plst.at[pl.ds(soff, S_CHUNK)], pv)
            sb = buf.at[pl.ds(0, S_CHUNK)]
            pltpu.async_copy(new_k.at[jv], sb, sgk).wait()
            pltpu.async_copy(sb, out_k.at[pv], ssk).wait()
            pltpu.async_copy(new_v.at[jv], sb, sgv).wait()
            pltpu.async_copy(sb, out_v.at[pv], ssv).wait()
            return carry

        lax.fori_loop(0, ncv, sbody, 0)

        pltpu.sync_copy(zsrc, buf)
        zbase = KEEP + wid * TAIL_PER_TILE
        zh = []
        for z in range(17):
            zh.append(pltpu.async_copy(
                buf, out_k.at[pl.ds(zbase + z * ZROWS, ZROWS)], ssk))
            zh.append(pltpu.async_copy(
                buf, out_v.at[pl.ds(zbase + z * ZROWS, ZROWS)], ssv))
        zh.append(pltpu.async_copy(
            buf.at[pl.ds(0, 8)], out_k.at[pl.ds(zbase + 816, 8)], ssk))
        zh.append(pltpu.async_copy(
            buf.at[pl.ds(0, 8)], out_v.at[pl.ds(zbase + 816, 8)], ssv))
        for h in zh:
            h.wait()

    return k(mem_k2, mem_v2, new_k2, new_v2, sel_hbm, jl_hbm, pl_hbm,
             cnt_hbm, zero_hbm)


def _build_core_lists(p, surv, core):
    if core == 0:
        mask = surv & ((p < C0_LO) | ((p >= C0_HI_BASE) & (p < C0_HI_END)))
    else:
        mask = surv & ~((p < C0_LO) | ((p >= C0_HI_BASE) & (p < C0_HI_END)))
    order = jnp.argsort(~mask, stable=True)
    cnt = mask.sum().astype(jnp.int32)
    per = (cnt + 15) // 16
    s = jnp.arange(16, dtype=jnp.int32)[:, None]
    l = jnp.arange(S_NCHUNK * S_CHUNK, dtype=jnp.int32)[None, :]
    g = s * per + l
    valid = (l < per) & (g < cnt)
    j_g = order[jnp.clip(g, 0, order.shape[0] - 1)].astype(jnp.int32)
    jl = jnp.where(valid, j_g, 0)
    tile_id = core * 16 + s
    dump = KEEP + tile_id * TAIL_PER_TILE + l
    pt = jnp.where(valid, p[j_g], dump).astype(jnp.int32)
    ntile = jnp.clip(cnt - s[:, 0] * per, 0, per)
    nch = (ntile + S_CHUNK - 1) // S_CHUNK
    return jl.reshape(-1), pt.reshape(-1), nch.astype(jnp.int32)


def kernel(mem_k, mem_v, idx, new_k, new_v, scores):
    B = mem_k.shape[0]
    mem_k2 = mem_k.reshape(TCAP, D)
    mem_v2 = mem_v.reshape(TCAP, D)
    new_k2 = new_k.reshape(RECENT, D)
    new_v2 = new_v.reshape(RECENT, D)

    cand = scores[0, SINK:OLD_END]
    _, top_local = lax.top_k(cand, TOP_C)
    sel = jnp.sort(top_local).astype(jnp.int32) + SINK

    idx32 = idx.astype(jnp.int32)
    last = jnp.concatenate([idx32[1:] != idx32[:-1],
                            jnp.ones((1,), dtype=bool)])
    q = jnp.clip(jnp.searchsorted(sel, idx32), 0, TOP_C - 1)
    in_sel = sel[q] == idx32
    p = jnp.where(idx32 < SINK, idx32,
                  jnp.where(idx32 >= OLD_END,
                            idx32 - OLD_END + SINK + TOP_C,
                            jnp.where(in_sel, SINK + q, -1)))
    surv = last & (p >= 0)

    jl0, pl0, nch0 = _build_core_lists(p, surv, 0)
    jl1, pl1, nch1 = _build_core_lists(p, surv, 1)
    jl = jnp.concatenate([jl0, jl1])
    plst = jnp.concatenate([pl0, pl1])
    nch = jnp.concatenate([nch0, nch1])
    cnts = jnp.broadcast_to(nch[:, None], (32, 16)).reshape(-1)

    zero_hbm = jnp.zeros((ZROWS, D), jnp.float32)

    out_k2, out_v2 = _sc_compact(mem_k2, mem_v2, new_k2, new_v2,
                                 sel, jl, plst, cnts, zero_hbm)

    out_k = out_k2.reshape(B, TCAP, 16, 128)
    out_v = out_v2.reshape(B, TCAP, 16, 128)

    pos = jnp.arange(TCAP)
    protected_mask = ((pos >= SINK) & (pos < SINK + TOP_C))[None, :]
    protected_len = protected_mask.sum(axis=1).astype(jnp.int64)
    return out_k, out_v, protected_mask, protected_len


# speedup vs baseline: 2.3541x; 2.3541x over previous
import functools
import jax
import jax.numpy as jnp
from jax import lax
from jax.experimental import pallas as pl
from jax.experimental.pallas import tpu as pltpu
from jax.experimental.pallas import tpu_sc as plsc

SINK = 256
RECENT = 4096
TOP_C = 2048
TCAP = 32768
OLD_END = TCAP - RECENT
KEEP = SINK + TOP_C + RECENT
D = 2048
NTILES = 32
ROWS_PER_TILE = KEEP // NTILES
G_CHUNK = 40
G_NCHUNK = ROWS_PER_TILE // G_CHUNK
S_CHUNK = 32
S_NCHUNK = 8
CORE_SPLIT = KEEP // 2


def _sc_compact(mem_k2, mem_v2, new_k2, new_v2, msrc, jl_l, pl_l, cnt_l):
    mesh = plsc.VectorSubcoreMesh(core_axis_name="c", subcore_axis_name="s",
                                  num_cores=2, num_subcores=16)

    @functools.partial(
        pl.kernel,
        out_type=(
            jax.ShapeDtypeStruct((TCAP, D), jnp.float32),
            jax.ShapeDtypeStruct((TCAP, D), jnp.float32),
        ),
        mesh=mesh,
        scratch_types=[
            pltpu.VMEM((G_CHUNK,), jnp.int32),
            pltpu.VMEM((G_CHUNK, D), jnp.float32),
            pltpu.VMEM((S_CHUNK,), jnp.int32),
            pltpu.VMEM((S_CHUNK,), jnp.int32),
            pltpu.VMEM((16,), jnp.int32),
            pltpu.SemaphoreType.DMA,
        ],
    )
    def k(mem_k, mem_v, new_k, new_v, src, jl, plst, cnts,
          out_k, out_v,
          idx_v, buf, jv, pv, cntv, sem):
        core = lax.axis_index("c")
        sub = lax.axis_index("s")
        wid = core * 16 + sub

        for c in range(G_NCHUNK):
            base = wid * ROWS_PER_TILE + c * G_CHUNK
            pltpu.sync_copy(src.at[pl.ds(base, G_CHUNK)], idx_v)
            pltpu.async_copy(mem_k.at[idx_v], buf, sem).wait()
            pltpu.sync_copy(buf, out_k.at[pl.ds(base, G_CHUNK)])
            pltpu.async_copy(mem_v.at[idx_v], buf, sem).wait()
            pltpu.sync_copy(buf, out_v.at[pl.ds(base, G_CHUNK)])

        plsc.subcore_barrier()

        pltpu.sync_copy(cnts.at[pl.ds(wid * 16, 16)], cntv)
        ncv = cntv[...][0]

        def sbody(c, carry):
            soff = (wid * S_NCHUNK + c) * S_CHUNK
            pltpu.sync_copy(jl.at[pl.ds(soff, S_CHUNK)], jv)
            pltpu.sync_copy(plst.at[pl.ds(soff, S_CHUNK)], pv)
            sb = buf.at[pl.ds(0, S_CHUNK)]
            pltpu.async_copy(new_k.at[jv], sb, sem).wait()
            pltpu.async_copy(sb, out_k.at[pv], sem).wait()
            pltpu.async_copy(new_v.at[jv], sb, sem).wait()
            pltpu.async_copy(sb, out_v.at[pv], sem).wait()
            return carry

        lax.fori_loop(0, ncv, sbody, 0)

    return k(mem_k2, mem_v2, new_k2, new_v2, msrc, jl_l, pl_l, cnt_l)


def _zero_tail(out_k2, out_v2):
    zb = 256
    nblk = (TCAP - KEEP) // zb

    def body(ik, iv, ok, ov):
        ok[...] = jnp.zeros_like(ok)
        ov[...] = jnp.zeros_like(ov)

    return pl.pallas_call(
        body,
        grid=(nblk,),
        in_specs=[
            pl.BlockSpec(memory_space=pl.ANY),
            pl.BlockSpec(memory_space=pl.ANY),
        ],
        out_specs=[
            pl.BlockSpec((zb, D), lambda b: (KEEP // zb + b, 0)),
            pl.BlockSpec((zb, D), lambda b: (KEEP // zb + b, 0)),
        ],
        out_shape=[
            jax.ShapeDtypeStruct((TCAP, D), jnp.float32),
            jax.ShapeDtypeStruct((TCAP, D), jnp.float32),
        ],
        input_output_aliases={0: 0, 1: 1},
    )(out_k2, out_v2)


def _build_core_lists(p, surv, core):
    if core == 0:
        mask = surv & (p < CORE_SPLIT)
    else:
        mask = surv & (p >= CORE_SPLIT)
    order = jnp.argsort(~mask, stable=True)
    cnt = mask.sum().astype(jnp.int32)
    per = (cnt + 15) // 16
    s = jnp.arange(16, dtype=jnp.int32)[:, None]
    l = jnp.arange(S_NCHUNK * S_CHUNK, dtype=jnp.int32)[None, :]
    g = s * per + l
    valid = (l < per) & (g < cnt)
    j_g = order[jnp.clip(g, 0, order.shape[0] - 1)].astype(jnp.int32)
    jl = jnp.where(valid, j_g, 0)
    tile_id = core * 16 + s
    dump = KEEP + tile_id * (S_NCHUNK * S_CHUNK) + l
    pt = jnp.where(valid, p[j_g], dump).astype(jnp.int32)
    ntile = jnp.clip(cnt - s[:, 0] * per, 0, per)
    nch = (ntile + S_CHUNK - 1) // S_CHUNK
    return jl.reshape(-1), pt.reshape(-1), nch.astype(jnp.int32)


def kernel(mem_k, mem_v, idx, new_k, new_v, scores):
    B = mem_k.shape[0]
    mem_k2 = mem_k.reshape(TCAP, D)
    mem_v2 = mem_v.reshape(TCAP, D)
    new_k2 = new_k.reshape(RECENT, D)
    new_v2 = new_v.reshape(RECENT, D)

    cand = scores[0, SINK:OLD_END]
    _, top_local = lax.top_k(cand, TOP_C)
    sel = jnp.sort(top_local).astype(jnp.int32) + SINK

    msrc = jnp.concatenate([
        jnp.arange(0, SINK, dtype=jnp.int32),
        sel,
        jnp.arange(OLD_END, TCAP, dtype=jnp.int32),
    ])

    idx32 = idx.astype(jnp.int32)
    last = jnp.concatenate([idx32[1:] != idx32[:-1],
                            jnp.ones((1,), dtype=bool)])
    q = jnp.clip(jnp.searchsorted(sel, idx32), 0, TOP_C - 1)
    in_sel = sel[q] == idx32
    p = jnp.where(idx32 < SINK, idx32,
                  jnp.where(idx32 >= OLD_END,
                            idx32 - OLD_END + SINK + TOP_C,
                            jnp.where(in_sel, SINK + q, -1)))
    surv = last & (p >= 0)

    jl0, pl0, nch0 = _build_core_lists(p, surv, 0)
    jl1, pl1, nch1 = _build_core_lists(p, surv, 1)
    jl = jnp.concatenate([jl0, jl1])
    plst = jnp.concatenate([pl0, pl1])
    nch = jnp.concatenate([nch0, nch1])
    cnts = jnp.broadcast_to(nch[:, None], (32, 16)).reshape(-1)

    out_k2, out_v2 = _sc_compact(mem_k2, mem_v2, new_k2, new_v2,
                                 msrc, jl, plst, cnts)
    out_k2, out_v2 = _zero_tail(out_k2, out_v2)

    out_k = out_k2.reshape(B, TCAP, 16, 128)
    out_v = out_v2.reshape(B, TCAP, 16, 128)

    pos = jnp.arange(TCAP)
    protected_mask = ((pos >= SINK) & (pos < SINK + TOP_C))[None, :]
    protected_len = protected_mask.sum(axis=1).astype(jnp.int64)
    return out_k, out_v, protected_mask, protected_len
